# baseline (speedup 1.0000x reference)
import functools

import jax
import jax.numpy as jnp
from jax import lax
from jax.experimental import pallas as pl
from jax.experimental.pallas import tpu as pltpu
from jax.experimental.pallas import tpu_sc as plsc

ROWS = 16384
COLS = 2048
NC = 2
NS = 16
L = 16
NW = NC * NS

SC_ROWS = 10240
RPW = SC_ROWS // NW
CHUNK = 8
STEPS = CHUNK * COLS // (16 * L)
NACC = 8
NSLOTS = 3

TC_ROWS = ROWS - SC_ROWS
BLOCK_ROWS = 512
TC_GRID = TC_ROWS // BLOCK_ROWS
TC_BLOCK_OFF = SC_ROWS // BLOCK_ROWS


def _chunk_sum(eb, ob, accs):
    def step(s, accs):
        r = s // (STEPS // CHUNK)
        cb = s % (STEPS // CHUNK)
        accs = list(accs)
        for k in range(16):
            off = cb * 256 + k * L
            ve = eb[r, pl.ds(off, L)]
            vo = ob[r, pl.ds(off, L)]
            d = ve - vo
            accs[k % NACC] = accs[k % NACC] + d * d
        return tuple(accs)

    return lax.fori_loop(0, STEPS, step, tuple(accs))


def _sc_body(e_hbm, o_hbm, m_hbm, out_hbm,
             m_v, lane_v, idx_v, e0, o0, e1, o1, e2, o2, res_v,
             sem0, sem1, sem2):
    cid = lax.axis_index("c")
    sid = lax.axis_index("s")
    wid = sid * NC + cid
    base = wid * RPW

    pltpu.sync_copy(m_hbm.at[pl.ds(wid * (RPW // 4), RPW // 4)], m_v)

    sent = jnp.full((L,), base, jnp.int32)

    REG = RPW // L
    DUMP = RPW
    lane_id = lax.iota(jnp.int32, L)
    lane_base = lane_id * REG
    cnt = jnp.zeros((L,), jnp.int32)
    for j in range(RPW // (4 * L)):
        mword = m_v[pl.ds(j * L, L)]
        for b in range(4):
            mb = ((mword >> (8 * b)) & 0xFF) > 0
            rows = base + j * 4 * L + lane_id * 4 + b
            pos = jnp.where(mb, lane_base + cnt, DUMP)
            plsc.store_scatter(lane_v, [pos], rows)
            cnt = cnt + jnp.where(mb, 1, 0)

    n = jnp.int32(0)
    off = jnp.zeros((L,), jnp.int32)
    for l in range(L):
        cl = cnt[l]
        off = off + jnp.where(lane_id > l, cl, 0)
        n = n + cl

    for t in range(REG):
        v = plsc.load_gather(lane_v, [lane_base + t])
        dst = jnp.where(t < cnt, off + t, DUMP)
        plsc.store_scatter(idx_v, [dst], v)

    plsc.store_scatter(idx_v, [n + lane_id], sent)
    plsc.store_scatter(idx_v, [n + L + lane_id], sent)

    ngroups = (n + NSLOTS * CHUNK - 1) // (NSLOTS * CHUNK)
    nch = ngroups * NSLOTS

    slots = ((e0, o0, sem0), (e1, o1, sem1), (e2, o2, sem2))[:NSLOTS]

    def fire(g, eb, ob, sem):
        iv = idx_v.at[pl.ds(g * CHUNK, CHUNK)]
        pltpu.make_async_copy(e_hbm.at[iv], eb, sem).start()
        pltpu.make_async_copy(o_hbm.at[iv], ob, sem).start()

    def drain(eb, ob, sem):
        iv = idx_v.at[pl.ds(0, CHUNK)]
        pltpu.make_async_copy(e_hbm.at[iv], eb, sem).wait()
        pltpu.make_async_copy(o_hbm.at[iv], ob, sem).wait()

    for s in range(NSLOTS):
        @pl.when(s < nch)
        def _prime(s=s):
            fire(s, *slots[s])

    zero = jnp.zeros((L,), jnp.float32)
    accs0 = tuple(zero for _ in range(NACC))

    def group_body(p, accs):
        for s in range(NSLOTS):
            eb, ob, sem = slots[s]
            g = NSLOTS * p + s
            drain(eb, ob, sem)
            accs = _chunk_sum(eb, ob, accs)

            @pl.when(g + NSLOTS < nch)
            def _refire():
                fire(g + NSLOTS, eb, ob, sem)

        return accs

    accs = lax.fori_loop(0, ngroups, group_body, accs0)
    tot_v = accs[0]
    for a in accs[1:]:
        tot_v = tot_v + a

    pltpu.sync_copy(e_hbm.at[base], e0.at[0])
    pltpu.sync_copy(o_hbm.at[base], o0.at[0])

    def srow_step(s, acc):
        for k in range(16):
            off = s * 256 + k * L
            d = e0[0, pl.ds(off, L)] - o0[0, pl.ds(off, L)]
            acc = acc + d * d
        return acc

    sacc = lax.fori_loop(0, COLS // (16 * L), srow_step, zero)
    npad = nch * CHUNK - n
    tot_v = tot_v - sacc * npad.astype(jnp.float32)

    res_v[0, :] = tot_v
    res_v[1, :] = jnp.full((L,), n.astype(jnp.float32))
    pltpu.sync_copy(res_v, out_hbm.at[wid])


@functools.lru_cache(maxsize=1)
def _build_sc_loss():
    return functools.partial(
        pl.kernel,
        out_type=jax.ShapeDtypeStruct((NW, 2, L), jnp.float32),
        mesh=plsc.VectorSubcoreMesh(core_axis_name="c", subcore_axis_name="s",
                                    num_cores=NC, num_subcores=NS),
        compiler_params=pltpu.CompilerParams(needs_layout_passes=False),
        scratch_types=[
            pltpu.VMEM((RPW // 4,), jnp.int32),
            pltpu.VMEM((RPW + 8,), jnp.int32),
            pltpu.VMEM((RPW + 2 * L,), jnp.int32),
            pltpu.VMEM((CHUNK, COLS), jnp.float32),
            pltpu.VMEM((CHUNK, COLS), jnp.float32),
            pltpu.VMEM((CHUNK, COLS), jnp.float32),
            pltpu.VMEM((CHUNK, COLS), jnp.float32),
            pltpu.VMEM((CHUNK, COLS), jnp.float32),
            pltpu.VMEM((CHUNK, COLS), jnp.float32),
            pltpu.VMEM((2, L), jnp.float32),
            pltpu.SemaphoreType.DMA,
            pltpu.SemaphoreType.DMA,
            pltpu.SemaphoreType.DMA,
        ],
    )(_sc_body)


def _tc_body(e_ref, o_ref, m_ref, out_ref, acc_ref, cnt_ref):
    i = pl.program_id(0)

    @pl.when(i == 0)
    def _init():
        acc_ref[0, 0] = 0.0
        cnt_ref[0, 0] = 0.0

    d = e_ref[...] - o_ref[...]
    m = m_ref[...]
    sq = d * d * m
    acc_ref[0, 0] += jnp.sum(sq)
    cnt_ref[0, 0] += jnp.sum(m)

    @pl.when(i == TC_GRID - 1)
    def _fin():
        out_ref[0, 0] = acc_ref[0, 0]
        out_ref[0, 1] = cnt_ref[0, 0]


def _tc_partial(e2, o2, m2):
    return pl.pallas_call(
        _tc_body,
        grid=(TC_GRID,),
        in_specs=[
            pl.BlockSpec((BLOCK_ROWS, COLS), lambda i: (i + TC_BLOCK_OFF, 0)),
            pl.BlockSpec((BLOCK_ROWS, COLS), lambda i: (i + TC_BLOCK_OFF, 0)),
            pl.BlockSpec((BLOCK_ROWS, 1), lambda i: (i + TC_BLOCK_OFF, 0)),
        ],
        out_specs=pl.BlockSpec((1, 2), lambda i: (0, 0),
                               memory_space=pltpu.SMEM),
        out_shape=jax.ShapeDtypeStruct((1, 2), jnp.float32),
        scratch_shapes=[
            pltpu.SMEM((1, 1), jnp.float32),
            pltpu.SMEM((1, 1), jnp.float32),
        ],
    )(e2, o2, m2)


@jax.jit
def _masked_mse(e2, o2, m1, m2):
    parts = _build_sc_loss()(e2, o2, m1)
    tc = _tc_partial(e2, o2, m2)
    total = jnp.sum(parts[:, 0, :]) + tc[0, 0]
    count = jnp.sum(parts[:, 1, 0]) + tc[0, 1]
    return total / jnp.maximum(count * float(COLS), 1.0)


def kernel(estimate, output, mask):
    e2 = estimate.reshape(ROWS, COLS)
    o2 = output.reshape(ROWS, COLS)
    mbytes = mask.reshape(ROWS).view(jnp.int8)
    m1 = lax.bitcast_convert_type(mbytes.reshape(ROWS // 4, 4), jnp.int32)
    m2 = mask.reshape(ROWS, 1).astype(jnp.float32)
    return _masked_mse(e2, o2, m1, m2)


# speedup vs baseline: 1.1861x; 1.1861x over previous
import functools

import jax
import jax.numpy as jnp
from jax import lax
from jax.experimental import pallas as pl
from jax.experimental.pallas import tpu as pltpu
from jax.experimental.pallas import tpu_sc as plsc

ROWS = 16384
COLS = 2048
NC = 2
NS = 16
L = 16
NW = NC * NS
RPW = ROWS // NW
CHUNK = 8
STEPS = CHUNK * COLS // (16 * L)
NACC = 8
NSLOTS = 3


def _chunk_sum(eb, ob, accs):
    def step(s, accs):
        r = s // (STEPS // CHUNK)
        cb = s % (STEPS // CHUNK)
        accs = list(accs)
        for k in range(16):
            off = cb * 256 + k * L
            ve = eb[r, pl.ds(off, L)]
            vo = ob[r, pl.ds(off, L)]
            d = ve - vo
            accs[k % NACC] = accs[k % NACC] + d * d
        return tuple(accs)

    return lax.fori_loop(0, STEPS, step, tuple(accs))


def _sc_body(e_hbm, o_hbm, m_hbm, out_hbm,
             m_v, lane_v, idx_v, e0, o0, e1, o1, e2, o2, res_v,
             sem0, sem1, sem2):
    cid = lax.axis_index("c")
    sid = lax.axis_index("s")
    wid = sid * NC + cid
    base = wid * RPW

    pltpu.sync_copy(m_hbm.at[pl.ds(base, RPW)], m_v)

    sent = jnp.full((L,), base, jnp.int32)

    REG = RPW // L
    DUMP = RPW
    lane_id = lax.iota(jnp.int32, L)
    lane_base = lane_id * REG

    def compact(j, cnt):
        mword = plsc.bitcast(m_v[pl.ds(j * 4 * L, 4 * L)], jnp.int32)
        for b in range(4):
            mb = ((mword >> (8 * b)) & 0xFF) > 0
            rows = base + j * 4 * L + lane_id * 4 + b
            pos = jnp.where(mb, lane_base + cnt, DUMP)
            plsc.store_scatter(lane_v, [pos], rows)
            cnt = cnt + jnp.where(mb, 1, 0)
        return cnt

    cnt = lax.fori_loop(0, RPW // (4 * L), compact,
                        jnp.zeros((L,), jnp.int32))

    n = jnp.int32(0)
    off = jnp.zeros((L,), jnp.int32)
    for l in range(L):
        cl = cnt[l]
        off = off + jnp.where(lane_id > l, cl, 0)
        n = n + cl

    def merge(t, _):
        v = plsc.load_gather(lane_v, [lane_base + t])
        dst = jnp.where(t < cnt, off + t, DUMP)
        plsc.store_scatter(idx_v, [dst], v)
        return 0

    lax.fori_loop(0, REG, merge, 0)

    plsc.store_scatter(idx_v, [n + lane_id], sent)
    plsc.store_scatter(idx_v, [n + L + lane_id], sent)

    ngroups = (n + NSLOTS * CHUNK - 1) // (NSLOTS * CHUNK)
    nch = ngroups * NSLOTS

    slots = ((e0, o0, sem0), (e1, o1, sem1), (e2, o2, sem2))[:NSLOTS]

    def fire(g, eb, ob, sem):
        iv = idx_v.at[pl.ds(g * CHUNK, CHUNK)]
        pltpu.make_async_copy(e_hbm.at[iv], eb, sem).start()
        pltpu.make_async_copy(o_hbm.at[iv], ob, sem).start()

    def drain(eb, ob, sem):
        iv = idx_v.at[pl.ds(0, CHUNK)]
        pltpu.make_async_copy(e_hbm.at[iv], eb, sem).wait()
        pltpu.make_async_copy(o_hbm.at[iv], ob, sem).wait()

    for s in range(NSLOTS):
        @pl.when(s < nch)
        def _prime(s=s):
            fire(s, *slots[s])

    zero = jnp.zeros((L,), jnp.float32)
    accs0 = tuple(zero for _ in range(NACC))

    def group_body(p, accs):
        for s in range(NSLOTS):
            eb, ob, sem = slots[s]
            g = NSLOTS * p + s
            drain(eb, ob, sem)
            accs = _chunk_sum(eb, ob, accs)

            @pl.when(g + NSLOTS < nch)
            def _refire():
                fire(g + NSLOTS, eb, ob, sem)

        return accs

    accs = lax.fori_loop(0, ngroups, group_body, accs0)
    tot_v = accs[0]
    for a in accs[1:]:
        tot_v = tot_v + a

    pltpu.sync_copy(e_hbm.at[base], e0.at[0])
    pltpu.sync_copy(o_hbm.at[base], o0.at[0])

    def srow_step(s, acc):
        for k in range(16):
            off2 = s * 256 + k * L
            d = e0[0, pl.ds(off2, L)] - o0[0, pl.ds(off2, L)]
            acc = acc + d * d
        return acc

    sacc = lax.fori_loop(0, COLS // (16 * L), srow_step, zero)
    npad = nch * CHUNK - n
    tot_v = tot_v - sacc * npad.astype(jnp.float32)

    res_v[0, :] = tot_v
    res_v[1, :] = jnp.full((L,), n.astype(jnp.float32))
    pltpu.sync_copy(res_v, out_hbm.at[wid])


@functools.lru_cache(maxsize=1)
def _build_sc_loss():
    return functools.partial(
        pl.kernel,
        out_type=jax.ShapeDtypeStruct((NW, 2, L), jnp.float32),
        mesh=plsc.VectorSubcoreMesh(core_axis_name="c", subcore_axis_name="s",
                                    num_cores=NC, num_subcores=NS),
        compiler_params=pltpu.CompilerParams(needs_layout_passes=False),
        scratch_types=[
            pltpu.VMEM((RPW,), jnp.int8),
            pltpu.VMEM((RPW + 8,), jnp.int32),
            pltpu.VMEM((RPW + 2 * L,), jnp.int32),
            pltpu.VMEM((CHUNK, COLS), jnp.float32),
            pltpu.VMEM((CHUNK, COLS), jnp.float32),
            pltpu.VMEM((CHUNK, COLS), jnp.float32),
            pltpu.VMEM((CHUNK, COLS), jnp.float32),
            pltpu.VMEM((CHUNK, COLS), jnp.float32),
            pltpu.VMEM((CHUNK, COLS), jnp.float32),
            pltpu.VMEM((2, L), jnp.float32),
            pltpu.SemaphoreType.DMA,
            pltpu.SemaphoreType.DMA,
            pltpu.SemaphoreType.DMA,
        ],
    )(_sc_body)


@jax.jit
def _masked_mse(e2, o2, m1):
    parts = _build_sc_loss()(e2, o2, m1)
    total = jnp.sum(parts[:, 0, :])
    count = jnp.sum(parts[:, 1, 0])
    return total / jnp.maximum(count * float(COLS), 1.0)


def kernel(estimate, output, mask):
    e2 = estimate.reshape(ROWS, COLS)
    o2 = output.reshape(ROWS, COLS)
    m1 = mask.reshape(ROWS).view(jnp.int8)
    return _masked_mse(e2, o2, m1)


# speedup vs baseline: 1.2097x; 1.0199x over previous
import functools

import jax
import jax.numpy as jnp
from jax import lax
from jax.experimental import pallas as pl
from jax.experimental.pallas import tpu as pltpu
from jax.experimental.pallas import tpu_sc as plsc

ROWS = 16384
COLS = 2048
NC = 2
NS = 16
L = 16
NW = NC * NS
RPW = ROWS // NW
CHUNK = 8
STEPS = CHUNK * COLS // (16 * L)
NACC = 8
NSLOTS = 3


def _chunk_sum(eb, ob, accs):
    def step(s, accs):
        r = s // (STEPS // CHUNK)
        cb = s % (STEPS // CHUNK)
        accs = list(accs)
        for k in range(16):
            off = cb * 256 + k * L
            ve = eb[r, pl.ds(off, L)]
            vo = ob[r, pl.ds(off, L)]
            d = ve - vo
            accs[k % NACC] = accs[k % NACC] + d * d
        return tuple(accs)

    return lax.fori_loop(0, STEPS, step, tuple(accs))


def _sc_body(e_hbm, o_hbm, m_hbm, out_hbm,
             m_v, lane_v, idx_v, e0, o0, e1, o1, e2, o2, res_v,
             sem0, sem1, sem2):
    cid = lax.axis_index("c")
    sid = lax.axis_index("s")
    wid = sid * NC + cid
    base = wid * RPW

    pltpu.sync_copy(m_hbm.at[pl.ds(base, RPW)], m_v)

    sent = jnp.full((L,), base, jnp.int32)

    REG = RPW // L
    DUMP = RPW
    lane_id = lax.iota(jnp.int32, L)
    lane_base = lane_id * REG

    def compact(j, cnt):
        mb = m_v[pl.ds(j * L, L)] > 0
        rows = base + j * L + lane_id
        pos = jnp.where(mb, lane_base + cnt, DUMP)
        plsc.store_scatter(lane_v, [pos], rows)
        return cnt + jnp.where(mb, 1, 0)

    cnt = lax.fori_loop(0, RPW // L, compact, jnp.zeros((L,), jnp.int32))

    n = jnp.int32(0)
    off = jnp.zeros((L,), jnp.int32)
    for l in range(L):
        cl = cnt[l]
        off = off + jnp.where(lane_id > l, cl, 0)
        n = n + cl

    def merge(t, _):
        v = plsc.load_gather(lane_v, [lane_base + t])
        dst = jnp.where(t < cnt, off + t, DUMP)
        plsc.store_scatter(idx_v, [dst], v)
        return 0

    lax.fori_loop(0, REG, merge, 0)

    plsc.store_scatter(idx_v, [n + lane_id], sent)
    plsc.store_scatter(idx_v, [n + L + lane_id], sent)

    ngroups = (n + NSLOTS * CHUNK - 1) // (NSLOTS * CHUNK)
    nch = ngroups * NSLOTS

    slots = ((e0, o0, sem0), (e1, o1, sem1), (e2, o2, sem2))[:NSLOTS]

    def fire(g, eb, ob, sem):
        iv = idx_v.at[pl.ds(g * CHUNK, CHUNK)]
        pltpu.make_async_copy(e_hbm.at[iv], eb, sem).start()
        pltpu.make_async_copy(o_hbm.at[iv], ob, sem).start()

    def drain(eb, ob, sem):
        iv = idx_v.at[pl.ds(0, CHUNK)]
        pltpu.make_async_copy(e_hbm.at[iv], eb, sem).wait()
        pltpu.make_async_copy(o_hbm.at[iv], ob, sem).wait()

    for s in range(NSLOTS):
        @pl.when(s < nch)
        def _prime(s=s):
            fire(s, *slots[s])

    zero = jnp.zeros((L,), jnp.float32)
    accs0 = tuple(zero for _ in range(NACC))

    def group_body(p, accs):
        for s in range(NSLOTS):
            eb, ob, sem = slots[s]
            g = NSLOTS * p + s
            drain(eb, ob, sem)
            accs = _chunk_sum(eb, ob, accs)

            @pl.when(g + NSLOTS < nch)
            def _refire():
                fire(g + NSLOTS, eb, ob, sem)

        return accs

    accs = lax.fori_loop(0, ngroups, group_body, accs0)
    tot_v = accs[0]
    for a in accs[1:]:
        tot_v = tot_v + a

    pltpu.sync_copy(e_hbm.at[base], e0.at[0])
    pltpu.sync_copy(o_hbm.at[base], o0.at[0])

    def srow_step(s, acc):
        for k in range(16):
            off2 = s * 256 + k * L
            d = e0[0, pl.ds(off2, L)] - o0[0, pl.ds(off2, L)]
            acc = acc + d * d
        return acc

    sacc = lax.fori_loop(0, COLS // (16 * L), srow_step, zero)
    npad = nch * CHUNK - n
    tot_v = tot_v - sacc * npad.astype(jnp.float32)

    res_v[0, :] = tot_v
    res_v[1, :] = jnp.full((L,), n.astype(jnp.float32))
    pltpu.sync_copy(res_v, out_hbm.at[wid])


@functools.lru_cache(maxsize=1)
def _build_sc_loss():
    return functools.partial(
        pl.kernel,
        out_type=jax.ShapeDtypeStruct((NW, 2, L), jnp.float32),
        mesh=plsc.VectorSubcoreMesh(core_axis_name="c", subcore_axis_name="s",
                                    num_cores=NC, num_subcores=NS),
        compiler_params=pltpu.CompilerParams(needs_layout_passes=False),
        scratch_types=[
            pltpu.VMEM((RPW,), jnp.int32),
            pltpu.VMEM((RPW + 8,), jnp.int32),
            pltpu.VMEM((RPW + 2 * L,), jnp.int32),
            pltpu.VMEM((CHUNK, COLS), jnp.float32),
            pltpu.VMEM((CHUNK, COLS), jnp.float32),
            pltpu.VMEM((CHUNK, COLS), jnp.float32),
            pltpu.VMEM((CHUNK, COLS), jnp.float32),
            pltpu.VMEM((CHUNK, COLS), jnp.float32),
            pltpu.VMEM((CHUNK, COLS), jnp.float32),
            pltpu.VMEM((2, L), jnp.float32),
            pltpu.SemaphoreType.DMA,
            pltpu.SemaphoreType.DMA,
            pltpu.SemaphoreType.DMA,
        ],
    )(_sc_body)


@jax.jit
def _masked_mse(e2, o2, m1):
    parts = _build_sc_loss()(e2, o2, m1)
    total = jnp.sum(parts[:, 0, :])
    count = jnp.sum(parts[:, 1, 0])
    return total / jnp.maximum(count * float(COLS), 1.0)


def kernel(estimate, output, mask):
    e2 = estimate.reshape(ROWS, COLS)
    o2 = output.reshape(ROWS, COLS)
    m1 = mask.reshape(ROWS).astype(jnp.int32)
    return _masked_mse(e2, o2, m1)
